# baseline (speedup 1.0000x reference)
import jax
import jax.numpy as jnp
from jax.experimental import pallas as pl

_B, _N, _F_ATOM, _F_PHYS = 1024, 100, 38, 3
_F_TOT = _F_ATOM + _F_PHYS
_SPLIT = 4
_bB = 128


def _head(x, Wr, br, Wc, bc, W1, b1, W5, b5, W6, b6, W7h, W7p, b7):
    xbar = jnp.sum(x, axis=1) * (1.0 / _N)
    xb = xbar[:, :_F_ATOM]
    phys = x[:, 0, _F_ATOM:]
    h = jax.nn.relu(jnp.dot(xb, Wr, preferred_element_type=jnp.float32) + br)
    g = jax.nn.relu(jnp.dot(h, Wc, preferred_element_type=jnp.float32)
                    + bc) * float(_N)
    d1 = jax.nn.relu(jnp.dot(g, W1, preferred_element_type=jnp.float32) + b1)
    d5 = jnp.dot(d1, W5, preferred_element_type=jnp.float32) + b5
    mv = jnp.dot(d5, W6, preferred_element_type=jnp.float32) + b6
    return mv * W7h[0, 0] + jnp.dot(phys, W7p,
                                    preferred_element_type=jnp.float32) + b7


def _fused_kernel(*refs):
    x_refs = refs[:_SPLIT]
    (Wr_ref, br_ref, Wc_ref, bc_ref, W1_ref, b1_ref, W5_ref, b5_ref,
     W6_ref, b6_ref, W7h_ref, W7p_ref, b7_ref, out_ref) = refs[_SPLIT:]
    for k in range(_SPLIT):
        res = _head(x_refs[k][...], Wr_ref[...], br_ref[...], Wc_ref[...],
                    bc_ref[...], W1_ref[...], b1_ref[...], W5_ref[...],
                    b5_ref[...], W6_ref[...], b6_ref[...], W7h_ref[...],
                    W7p_ref[...], b7_ref[...])
        out_ref[k * _bB:(k + 1) * _bB, :] = res


def _x_spec(k):
    return pl.BlockSpec((_bB, _N, _F_TOT), lambda i, k=k: (i * _SPLIT + k, 0, 0))


def kernel(inputs, W_rule, b_rule, W_conv, b_conv, W1, b1, W5, b5, W6, b6,
           W7, b7):
    B, N, F_tot = inputs.shape
    R = W_rule.shape[1]

    grid = (B // (_SPLIT * _bB),)
    out = pl.pallas_call(
        _fused_kernel,
        grid=grid,
        in_specs=[_x_spec(k) for k in range(_SPLIT)] + [
            pl.BlockSpec(W_rule.shape, lambda i: (0, 0)),
            pl.BlockSpec((1, R), lambda i: (0, 0)),
            pl.BlockSpec(W_conv.shape, lambda i: (0, 0)),
            pl.BlockSpec((1, W_conv.shape[1]), lambda i: (0, 0)),
            pl.BlockSpec(W1.shape, lambda i: (0, 0)),
            pl.BlockSpec((1, W1.shape[1]), lambda i: (0, 0)),
            pl.BlockSpec(W5.shape, lambda i: (0, 0)),
            pl.BlockSpec((1, W5.shape[1]), lambda i: (0, 0)),
            pl.BlockSpec(W6.shape, lambda i: (0, 0)),
            pl.BlockSpec((1, 1), lambda i: (0, 0)),
            pl.BlockSpec((1, 1), lambda i: (0, 0)),
            pl.BlockSpec((_F_PHYS, 1), lambda i: (0, 0)),
            pl.BlockSpec((1, 1), lambda i: (0, 0)),
        ],
        out_specs=pl.BlockSpec((_SPLIT * _bB, 1), lambda i: (i, 0)),
        out_shape=jax.ShapeDtypeStruct((B, 1), jnp.float32),
    )(*([inputs] * _SPLIT), W_rule, b_rule.reshape(1, -1), W_conv,
      b_conv.reshape(1, -1), W1, b1.reshape(1, -1), W5, b5.reshape(1, -1),
      W6, b6.reshape(1, -1), W7[0:1, :], W7[1:4, :], b7.reshape(1, -1))
    return out
